# baseline (speedup 1.0000x reference)
import jax
import jax.numpy as jnp
from jax import lax
from jax.experimental import pallas as pl
from jax.experimental.pallas import tpu as pltpu
from jax.experimental.pallas import tpu_sc as plsc

VOCAB = 100000
DEMB = 64
BATCH = 16384
SEQ = 50
EPS = 1e-5

NC = 2
NS = 16
NW = NC * NS
L = 16

N = BATCH * SEQ
PER_W = N // NW
CHUNK = 512
IDX_W = 128
IDX_ROWS = CHUNK // IDX_W
N_CHUNKS = PER_W // CHUNK

TBLK = 1000


def _ln_table_body(table_ref, gamma_ref, beta_ref, out_ref):
    x = table_ref[...]
    mean = jnp.mean(x, axis=-1, keepdims=True)
    var = jnp.mean((x - mean) * (x - mean), axis=-1, keepdims=True)
    normed = (x - mean) * lax.rsqrt(var + EPS)
    out_ref[...] = normed * gamma_ref[...] + beta_ref[...]


def _gather_body(table_hbm, idx_hbm, out_hbm, idx_v, rows_v, sem_g, sem_s):
    wid = lax.axis_index("s") * NC + lax.axis_index("c")
    idx_row0 = wid * (PER_W // IDX_W)
    out0 = wid * PER_W

    def stage_idx(g, b):
        pltpu.sync_copy(idx_hbm.at[pl.ds(idx_row0 + g * IDX_ROWS, IDX_ROWS)],
                        idx_v.at[b])

    def gather_descs(b, make_only):
        descs = []
        for j in range(IDX_ROWS):
            src = table_hbm.at[idx_v.at[b].at[j]]
            dst = rows_v.at[b].at[pl.ds(j * IDX_W, IDX_W)]
            if make_only:
                descs.append(pltpu.make_async_copy(src, dst, sem_g))
            else:
                descs.append(pltpu.async_copy(src, dst, sem_g))
        return descs

    def wait_store(g, b):
        pltpu.make_async_copy(
            rows_v.at[b],
            out_hbm.at[pl.ds(out0 + g * CHUNK, CHUNK)], sem_s).wait()

    stage_idx(0, 0)
    gather_descs(0, make_only=False)

    def pair_step(k, _):
        for b in range(2):
            g = 2 * k + b
            nb = 1 - b

            @pl.when(g >= 1)
            def _wait_prev_store():
                wait_store(g - 1, nb)

            @pl.when(g + 1 < N_CHUNKS)
            def _prefetch_next():
                stage_idx(g + 1, nb)
                gather_descs(nb, make_only=False)

            for d in gather_descs(b, make_only=True):
                d.wait()
            pltpu.async_copy(rows_v.at[b],
                             out_hbm.at[pl.ds(out0 + g * CHUNK, CHUNK)],
                             sem_s)
        return 0

    lax.fori_loop(0, N_CHUNKS // 2, pair_step, 0)
    wait_store(N_CHUNKS - 1, (N_CHUNKS - 1) % 2)


@jax.jit
def _run(goal_input, table, gamma, beta):
    normed_table = pl.pallas_call(
        _ln_table_body,
        grid=(VOCAB // TBLK,),
        in_specs=[
            pl.BlockSpec((TBLK, DEMB), lambda i: (i, 0)),
            pl.BlockSpec((1, DEMB), lambda i: (0, 0)),
            pl.BlockSpec((1, DEMB), lambda i: (0, 0)),
        ],
        out_specs=pl.BlockSpec((TBLK, DEMB), lambda i: (i, 0)),
        out_shape=jax.ShapeDtypeStruct((VOCAB, DEMB), jnp.float32),
    )(table, gamma.reshape(1, DEMB), beta.reshape(1, DEMB))

    idx = goal_input.reshape(N // IDX_W, IDX_W)
    mesh = plsc.VectorSubcoreMesh(core_axis_name="c", subcore_axis_name="s")
    out = pl.kernel(
        _gather_body,
        out_type=jax.ShapeDtypeStruct((N, DEMB), jnp.float32),
        mesh=mesh,
        scratch_types=[
            pltpu.VMEM((2, IDX_ROWS, IDX_W), jnp.int32),
            pltpu.VMEM((2, CHUNK, DEMB), jnp.float32),
            pltpu.SemaphoreType.DMA,
            pltpu.SemaphoreType.DMA,
        ],
        compiler_params=pltpu.CompilerParams(
            needs_layout_passes=False, use_tc_tiling_on_sc=False),
    )(normed_table, idx)
    return out.reshape(BATCH, SEQ, DEMB)


def kernel(goal_input, table, gamma, beta):
    return _run(goal_input, table, gamma, beta)
